# baseline (speedup 1.0000x reference)
import functools

import jax
import jax.numpy as jnp
import numpy as np
from jax import lax
from jax.experimental import pallas as pl
from jax.experimental.pallas import tpu as pltpu
from jax.experimental.pallas import tpu_sc as plsc

_FIELD_DIMS = [1000] * 26
_NUM_FIELDS = len(_FIELD_DIMS)
_TOTAL = int(sum(_FIELD_DIMS))
_EMBED_DIM = 32
_BATCH = 4096
_HOT = _FIELD_DIMS[0]
_ROW_STRIDE = _HOT

_NUM_CORES = 2
_NUM_SUBCORES = 16
_NUM_WORKERS = _NUM_CORES * _NUM_SUBCORES
_N = _BATCH * _NUM_FIELDS
_PER_W = _N // _NUM_WORKERS
_CHUNK = 128
_NCHUNK = _PER_W // _CHUNK
_LANES = 16


def _body(x_hbm, tab_hbm, out_hbm, idx_v, rows_v, gsem, osem):
    wid = lax.axis_index("s") * _NUM_CORES + lax.axis_index("c")
    base = wid * _PER_W

    pltpu.sync_copy(x_hbm.at[pl.ds(base, _PER_W)], idx_v)

    def add_off(j, _):
        s = j * _LANES
        pos = s + lax.iota(jnp.int32, _LANES)
        fid = pos % _NUM_FIELDS
        idx_v[pl.ds(s, _LANES)] = idx_v[pl.ds(s, _LANES)] + fid * _ROW_STRIDE
        return _

    lax.fori_loop(0, _PER_W // _LANES, add_off, None)

    gathers = []
    for j in range(_NCHUNK):
        gathers.append(pltpu.async_copy(
            tab_hbm.at[idx_v.at[pl.ds(j * _CHUNK, _CHUNK)]],
            rows_v.at[pl.ds(j * _CHUNK, _CHUNK)],
            gsem,
        ))
    outs = []
    for j in range(_NCHUNK):
        gathers[j].wait()
        outs.append(pltpu.async_copy(
            rows_v.at[pl.ds(j * _CHUNK, _CHUNK)],
            out_hbm.at[pl.ds(base + j * _CHUNK, _CHUNK)],
            osem,
        ))
    for o in outs:
        o.wait()


@jax.jit
def kernel(x, tables):
    blocks = tables.reshape(_NUM_FIELDS, _NUM_FIELDS, _HOT, _EMBED_DIM)
    hot = blocks[jnp.arange(_NUM_FIELDS), jnp.arange(_NUM_FIELDS)]
    tab_flat = hot.reshape(_NUM_FIELDS * _HOT, _EMBED_DIM)
    x_flat = x.reshape(_N)
    mesh = plsc.VectorSubcoreMesh(core_axis_name="c", subcore_axis_name="s")
    out = pl.kernel(
        _body,
        out_type=jax.ShapeDtypeStruct((_N, _EMBED_DIM), jnp.float32),
        mesh=mesh,
        scratch_types=[
            pltpu.VMEM((_PER_W,), jnp.int32),
            pltpu.VMEM((_PER_W, _EMBED_DIM), jnp.float32),
            pltpu.SemaphoreType.DMA,
            pltpu.SemaphoreType.DMA,
        ],
        compiler_params=pltpu.CompilerParams(use_tc_tiling_on_sc=False),
    )(x_flat, tab_flat)
    return out.reshape(_BATCH, _NUM_FIELDS, _EMBED_DIM)


# speedup vs baseline: 2.2384x; 2.2384x over previous
import functools

import jax
import jax.numpy as jnp
import numpy as np
from jax import lax
from jax.experimental import pallas as pl
from jax.experimental.pallas import tpu as pltpu
from jax.experimental.pallas import tpu_sc as plsc

_FIELD_DIMS = [1000] * 26
_NUM_FIELDS = len(_FIELD_DIMS)
_TOTAL = int(sum(_FIELD_DIMS))
_EMBED_DIM = 32
_BATCH = 4096
_HOT = _FIELD_DIMS[0]
_ROW_STRIDE = _HOT

_NUM_CORES = 2
_NUM_SUBCORES = 16
_NUM_WORKERS = _NUM_CORES * _NUM_SUBCORES
_N = _BATCH * _NUM_FIELDS
_PER_W = _N // _NUM_WORKERS
_CHUNK = 128
_NCHUNK = _PER_W // _CHUNK
_LANES = 16
_PROW = _CHUNK * _EMBED_DIM // 128


def _body(x_hbm, tab_hbm, out_hbm, idx_v, rows_v, p_v, gsem, osem):
    wid = lax.axis_index("s") * _NUM_CORES + lax.axis_index("c")
    base = wid * _PER_W

    pltpu.sync_copy(x_hbm.at[pl.ds(base, _PER_W)], idx_v)

    def add_off(j, carry):
        s = j * _LANES
        pos = base + s + lax.iota(jnp.int32, _LANES)
        fid = lax.shift_right_logical(pos, 12)
        idx_v[pl.ds(s, _LANES)] = idx_v[pl.ds(s, _LANES)] + fid * _ROW_STRIDE
        return carry

    lax.fori_loop(0, _PER_W // _LANES, add_off, None)

    gathers = []
    for j in range(_NCHUNK):
        gathers.append(pltpu.async_copy(
            tab_hbm.at[idx_v.at[pl.ds(j * _CHUNK, _CHUNK)]],
            rows_v.at[pl.ds(j * _CHUNK, _CHUNK)],
            gsem,
        ))
    for g in gathers:
        g.wait()

    def repack(j, carry):
        buf = j % 2
        g0 = j * _CHUNK

        def drain(_):
            pltpu.make_async_copy(
                x_hbm.at[pl.ds(0, _PROW * 32)],
                p_v.at[0],
                osem,
            ).wait()
            return ()

        lax.cond(j >= 2, drain, lambda _: (), None)
        for r in range(_PROW):
            for c in range(8):
                flat = r * 128 + c * 16
                p_v[buf, r, pl.ds(c * 16, _LANES)] = rows_v[
                    g0 + flat // _EMBED_DIM,
                    pl.ds(flat % _EMBED_DIM, _LANES),
                ]
        pltpu.async_copy(
            p_v.at[buf],
            out_hbm.at[pl.ds(base // 4 + j * _PROW, _PROW)],
            osem,
        )
        return carry

    lax.fori_loop(0, _NCHUNK, repack, None)
    for _ in range(2):
        pltpu.make_async_copy(
            x_hbm.at[pl.ds(0, _PROW * 32)],
            p_v.at[0],
            osem,
        ).wait()


@jax.jit
def kernel(x, tables):
    hot1d = jnp.concatenate([
        lax.slice(tables, (f, _HOT * f, 0), (f + 1, _HOT * (f + 1), _EMBED_DIM))
        .reshape(_HOT * _EMBED_DIM)
        for f in range(_NUM_FIELDS)
    ])
    tab_flat = hot1d.reshape(_NUM_FIELDS * _HOT, _EMBED_DIM)
    x_flat = jnp.transpose(x).reshape(_N)
    mesh = plsc.VectorSubcoreMesh(core_axis_name="c", subcore_axis_name="s")
    out = pl.kernel(
        _body,
        out_type=jax.ShapeDtypeStruct((_N * _EMBED_DIM // 128, 128),
                                      jnp.float32),
        mesh=mesh,
        scratch_types=[
            pltpu.VMEM((_PER_W,), jnp.int32),
            pltpu.VMEM((_PER_W, _EMBED_DIM), jnp.float32),
            pltpu.VMEM((2, _PROW, 128), jnp.float32),
            pltpu.SemaphoreType.DMA,
            pltpu.SemaphoreType.DMA,
        ],
        compiler_params=pltpu.CompilerParams(use_tc_tiling_on_sc=False),
    )(x_flat, tab_flat)
    return out.reshape(_NUM_FIELDS, _BATCH, _EMBED_DIM).transpose(1, 0, 2)


# speedup vs baseline: 2.6649x; 1.1905x over previous
import functools

import jax
import jax.numpy as jnp
import numpy as np
from jax import lax
from jax.experimental import pallas as pl
from jax.experimental.pallas import tpu as pltpu
from jax.experimental.pallas import tpu_sc as plsc

_FIELD_DIMS = [1000] * 26
_NUM_FIELDS = len(_FIELD_DIMS)
_TOTAL = int(sum(_FIELD_DIMS))
_EMBED_DIM = 32
_BATCH = 4096
_HOT = _FIELD_DIMS[0]
_ROW_STRIDE = _HOT

_NUM_CORES = 2
_NUM_SUBCORES = 16
_NUM_WORKERS = _NUM_CORES * _NUM_SUBCORES
_N = _BATCH * _NUM_FIELDS
_PER_W = _N // _NUM_WORKERS
_CHUNK = 128
_NCHUNK = _PER_W // _CHUNK
_LANES = 16


def _body(x_hbm, tab_hbm, out_hbm, idx_v, rows_v, gsem, osem):
    wid = lax.axis_index("s") * _NUM_CORES + lax.axis_index("c")
    base = wid * _PER_W

    pltpu.sync_copy(x_hbm.at[pl.ds(base, _PER_W)], idx_v)

    def add_off(j, _):
        s = j * _LANES
        pos = base + s + lax.iota(jnp.int32, _LANES)
        fid = lax.shift_right_logical(pos, 12)
        idx_v[pl.ds(s, _LANES)] = idx_v[pl.ds(s, _LANES)] + fid * _ROW_STRIDE
        return _

    lax.fori_loop(0, _PER_W // _LANES, add_off, None)

    gathers = []
    for j in range(_NCHUNK):
        gathers.append(pltpu.async_copy(
            tab_hbm.at[idx_v.at[pl.ds(j * _CHUNK, _CHUNK)]],
            rows_v.at[pl.ds(j * _CHUNK, _CHUNK)],
            gsem,
        ))
    outs = []
    for j in range(_NCHUNK):
        gathers[j].wait()
        pos = base + j * _CHUNK
        outs.append(pltpu.async_copy(
            rows_v.at[pl.ds(j * _CHUNK, _CHUNK)],
            out_hbm.at[pos // _BATCH, pl.ds(pos % _BATCH, _CHUNK),
                       pl.ds(0, _EMBED_DIM)],
            osem,
        ))
    for o in outs:
        o.wait()


@jax.jit
def kernel(x, tables):
    hot1d = jnp.concatenate([
        lax.slice(tables, (f, _HOT * f, 0), (f + 1, _HOT * (f + 1), _EMBED_DIM))
        .reshape(_HOT * _EMBED_DIM)
        for f in range(_NUM_FIELDS)
    ])
    tab_flat = hot1d.reshape(_NUM_FIELDS * _HOT, _EMBED_DIM)
    x_flat = jnp.transpose(x).reshape(_N)
    mesh = plsc.VectorSubcoreMesh(core_axis_name="c", subcore_axis_name="s")
    out = pl.kernel(
        _body,
        out_type=jax.ShapeDtypeStruct((_NUM_FIELDS, _BATCH, 4 * _EMBED_DIM), jnp.float32),
        mesh=mesh,
        scratch_types=[
            pltpu.VMEM((_PER_W,), jnp.int32),
            pltpu.VMEM((_PER_W, _EMBED_DIM), jnp.float32),
            pltpu.SemaphoreType.DMA,
            pltpu.SemaphoreType.DMA,
        ],
        compiler_params=pltpu.CompilerParams(use_tc_tiling_on_sc=False),
    )(x_flat, tab_flat)
    return out[:, :, : _EMBED_DIM].transpose(1, 0, 2)


# speedup vs baseline: 2.6668x; 1.0007x over previous
import jax
import jax.numpy as jnp
from jax import lax
from jax.experimental import pallas as pl
from jax.experimental.pallas import tpu as pltpu
from jax.experimental.pallas import tpu_sc as plsc

_FIELD_DIMS = [1000] * 26
_NUM_FIELDS = len(_FIELD_DIMS)
_TOTAL = int(sum(_FIELD_DIMS))
_EMBED_DIM = 32
_BATCH = 4096
_HOT = _FIELD_DIMS[0]
_ROW_STRIDE = _HOT

_NUM_CORES = 2
_NUM_SUBCORES = 16
_NUM_WORKERS = _NUM_CORES * _NUM_SUBCORES
_N = _BATCH * _NUM_FIELDS
_PER_W = _N // _NUM_WORKERS
_CHUNK = 128
_NCHUNK = _PER_W // _CHUNK
_LANES = 16


def _body(x_hbm, tab_hbm, out_hbm, idx_v, rows_v, gsem, osem):
    wid = lax.axis_index("s") * _NUM_CORES + lax.axis_index("c")
    base = wid * _PER_W

    pltpu.sync_copy(x_hbm.at[pl.ds(base, _PER_W)], idx_v)

    def add_off(j, _):
        s = j * _LANES
        pos = base + s + lax.iota(jnp.int32, _LANES)
        fid = lax.shift_right_logical(pos, 12)
        idx_v[pl.ds(s, _LANES)] = idx_v[pl.ds(s, _LANES)] + fid * _ROW_STRIDE
        return _

    lax.fori_loop(0, _PER_W // _LANES, add_off, None)

    gathers = []
    for j in range(_NCHUNK):
        gathers.append(pltpu.async_copy(
            tab_hbm.at[idx_v.at[pl.ds(j * _CHUNK, _CHUNK)]],
            rows_v.at[pl.ds(j * _CHUNK, _CHUNK)],
            gsem,
        ))
    outs = []
    for j in range(_NCHUNK):
        gathers[j].wait()
        pos = base + j * _CHUNK
        outs.append(pltpu.async_copy(
            rows_v.at[pl.ds(j * _CHUNK, _CHUNK)],
            out_hbm.at[pos // _BATCH, pl.ds(pos % _BATCH, _CHUNK),
                       pl.ds(0, _EMBED_DIM)],
            osem,
        ))
    for o in outs:
        o.wait()


@jax.jit
def kernel(x, tables):
    hot1d = jnp.concatenate([
        lax.slice(tables, (f, _HOT * f, 0), (f + 1, _HOT * (f + 1), _EMBED_DIM))
        .reshape(_HOT * _EMBED_DIM)
        for f in range(_NUM_FIELDS)
    ])
    tab_flat = hot1d.reshape(_NUM_FIELDS * _HOT, _EMBED_DIM)
    x_flat = jnp.transpose(x).reshape(_N)
    mesh = plsc.VectorSubcoreMesh(core_axis_name="c", subcore_axis_name="s")
    out = pl.kernel(
        _body,
        out_type=jax.ShapeDtypeStruct((_NUM_FIELDS, _BATCH, 4 * _EMBED_DIM), jnp.float32),
        mesh=mesh,
        scratch_types=[
            pltpu.VMEM((_PER_W,), jnp.int32),
            pltpu.VMEM((_PER_W, _EMBED_DIM), jnp.float32),
            pltpu.SemaphoreType.DMA,
            pltpu.SemaphoreType.DMA,
        ],
        compiler_params=pltpu.CompilerParams(use_tc_tiling_on_sc=False),
    )(x_flat, tab_flat)
    return out[:, :, : _EMBED_DIM].transpose(1, 0, 2)
